# baseline (speedup 1.0000x reference)
import jax
import jax.numpy as jnp
from jax import lax
from jax.experimental import pallas as pl
from jax.experimental.pallas import tpu as pltpu
from jax.experimental.pallas import tpu_sc as plsc

B = 8
N = 262144
D = H = W = 64
NBINS = D * H * W
NC = 2
NS = 16
L = 16
BPC = B // NC
NG = N // 128
GPT = NG // NS
PPT = N // NS
NGRP = PPT // L
BINS_PT = NBINS // NS
NROW = PPT // 128
ZB = 8192
SLOT = NS * 64
NSLOT = BPC * 2
SCALE = float(64.0 - 1e-06)
ONE_M_EPS = float(1.0 - 1e-06)
INF = float("inf")


def _perm(v, idx):
    return v.at[idx].get(mode="promise_in_bounds", unique_indices=False)


def _bf_min(v, lane):
    for sh in (8, 4, 2, 1):
        v = jnp.minimum(v, _perm(v, lane ^ sh))
    return v


def _bf_max(v, lane):
    for sh in (8, 4, 2, 1):
        v = jnp.maximum(v, _perm(v, lane ^ sh))
    return v


def _body(points_ref, grid_ref, mins_ref, ranges_ref,
          stage, idxbuf, ones, hslice, hslicep, zbuf, prow, pidx, pgridf, hist, mm,
          sem, sem2):
    c = lax.axis_index("c")
    s = lax.axis_index("s")
    lane = lax.broadcasted_iota(jnp.int32, (L,), 0)

    zeros16 = jnp.zeros((L,), jnp.float32)
    ones16 = jnp.full((L,), 1.0, jnp.float32)

    def fill(k, _):
        ones[pl.ds(k * L, L)] = ones16
        return 0

    def zfill(k, _):
        zbuf[pl.ds(k * L, L)] = zeros16
        return 0

    def pfill(k, _):
        hslicep[k >> 9, (k >> 3) & 63, pl.ds((k & 7) * L, L)] = zeros16
        return 0

    lax.fori_loop(0, 128 // L, fill, 0)
    lax.fori_loop(0, ZB // L, zfill, 0)
    lax.fori_loop(0, 2 * 64 * 8, pfill, 0)
    for q in range(BINS_PT // ZB):
        pltpu.sync_copy(zbuf, hist.at[pl.ds(s * BINS_PT + q * ZB, ZB)])
    mm_share = NSLOT * SLOT // NS
    pltpu.sync_copy(zbuf.at[pl.ds(0, mm_share)],
                    mm.at[pl.ds(s * mm_share, mm_share)])
    plsc.subcore_barrier()

    def stage_minmax_publish(bl):
        b = c * BPC + bl
        with jax.named_scope("stagein"):
            cps = [pltpu.async_copy(
                points_ref.at[cc, pl.ds(s * GPT, GPT), pl.ds(b, 1), :],
                stage.at[cc], sem2) for cc in range(3)]
            for cp in cps:
                cp.wait()

        def mm_body(r, carry):
            accs = list(carry)
            for d in range(8):
                col = d * L
                for cc in range(3):
                    v = stage[cc, r, 0, pl.ds(col, L)]
                    accs[cc] = jnp.minimum(accs[cc], v)
                    accs[3 + cc] = jnp.maximum(accs[3 + cc], v)
            return tuple(accs)

        init = (jnp.full((L,), INF), jnp.full((L,), INF), jnp.full((L,), INF),
                jnp.full((L,), -INF), jnp.full((L,), -INF), jnp.full((L,), -INF))
        with jax.named_scope("minmax"):
            acc6 = lax.fori_loop(0, GPT, mm_body, init)

        row = jnp.full((L,), INF)
        for cc in range(3):
            row = jnp.where(lane == cc, _bf_min(acc6[cc], lane), row)
            row = jnp.where(lane == 3 + cc, -_bf_max(acc6[3 + cc], lane), row)
        prow[...] = row
        pidx[...] = (bl * 2) * SLOT + s * 64 + lane
        pltpu.sync_copy(prow, mm.at[pidx], add=True)

    stage_minmax_publish(0)
    plsc.subcore_barrier()

    for bl in range(BPC):
        b = c * BPC + bl

        pltpu.sync_copy(mm.at[pl.ds((bl * 2) * SLOT, SLOT)], pgridf)
        acc = pgridf[pl.ds(0, L)]
        for t in range(1, NS):
            acc = jnp.minimum(acc, pgridf[pl.ds(t * 64, L)])
        mnb = [_perm(acc, jnp.full((L,), cc, jnp.int32)) for cc in range(3)]
        ngb = [_perm(acc, jnp.full((L,), 3 + cc, jnp.int32)) for cc in range(3)]
        rngb = [jnp.maximum(-ngb[cc] - mnb[cc], 1e-06) for cc in range(3)]
        invb = [1.0 / rngb[cc] for cc in range(3)]

        @pl.when(s == 0)
        def _():
            prow[...] = jnp.where(lane == 0, mnb[0],
                                  jnp.where(lane == 1, mnb[1],
                                            jnp.where(lane == 2, mnb[2], 0.0)))
            pltpu.sync_copy(prow, mins_ref.at[b])
            prow[...] = jnp.where(lane == 0, rngb[0],
                                  jnp.where(lane == 1, rngb[1],
                                            jnp.where(lane == 2, rngb[2], 0.0)))
            pltpu.sync_copy(prow, ranges_ref.at[b])

        def row_body(j, _):
            for d in range(8):
                col = d * L
                ii = []
                for cc in range(3):
                    xs = stage[cc, j, 0, pl.ds(col, L)]
                    v = (xs - mnb[cc]) * invb[cc]
                    v = jnp.minimum(jnp.maximum(v, 0.0), ONE_M_EPS) * SCALE
                    ii.append(v.astype(jnp.int32))
                linear = ii[0] * (H * W) + ii[1] * W + ii[2]
                idxbuf[j, pl.ds(col, L)] = linear
            pltpu.async_copy(ones, hist.at[idxbuf.at[j]], sem, add=True)
            return 0

        with jax.named_scope("idxsc"):
            lax.fori_loop(0, NROW, row_body, 0)

        if bl + 1 < BPC:
            stage_minmax_publish(bl + 1)

        with jax.named_scope("drain"):
            for j in range(NROW):
                pltpu.make_async_copy(ones, hist.at[idxbuf.at[0]], sem).wait()
        plsc.subcore_barrier()

        pltpu.sync_copy(hist.at[pl.ds(s * BINS_PT, BINS_PT)], hslice)
        for q in range(BINS_PT // ZB):
            pltpu.sync_copy(zbuf, hist.at[pl.ds(s * BINS_PT + q * ZB, ZB)])

        def max_body(r, m):
            for d in range(8):
                m = jnp.maximum(m, hslice[pl.ds(r * 128 + d * L, L)])
            return m

        with jax.named_scope("hmax"):
            m = lax.fori_loop(0, BINS_PT // 128, max_body, jnp.full((L,), -INF))
        prow[...] = jnp.where(lane == 0, -_bf_max(m, lane), INF)
        pidx[...] = (bl * 2 + 1) * SLOT + s * 64 + lane
        pltpu.sync_copy(prow, mm.at[pidx], add=True)
        plsc.subcore_barrier()

        pltpu.sync_copy(mm.at[pl.ds((bl * 2 + 1) * SLOT, SLOT)], pgridf)
        acc = pgridf[pl.ds(0, L)]
        for t in range(1, NS):
            acc = jnp.minimum(acc, pgridf[pl.ds(t * 64, L)])
        gmaxb = -_perm(acc, jnp.zeros((L,), jnp.int32))
        invn = 1.0 / jnp.maximum(gmaxb, 1.0)

        with jax.named_scope("norm"):
            cps = [None, None]
            for q in range(4):
                pb = q & 1
                if cps[pb] is not None:
                    cps[pb].wait()

                def normq(r8, _):
                    for rr in range(8):
                        r = r8 * 8 + rr
                        for d in range(4):
                            o = q * 4096 + r * 64 + d * L
                            hslicep[pb, r, pl.ds(d * L, L)] = (
                                hslice[pl.ds(o, L)] * invn)
                    return 0

                lax.fori_loop(0, 8, normq, 0)
                cps[pb] = pltpu.async_copy(
                    hslicep.at[pb],
                    grid_ref.at[b, pl.ds(s * 256 + q * 64, 64), :], sem2)
            for cp in cps:
                cp.wait()


@jax.jit
def _voxelize(points4):
    mesh = plsc.VectorSubcoreMesh(core_axis_name="c", subcore_axis_name="s",
                                  num_cores=NC, num_subcores=NS)
    f = pl.kernel(
        _body,
        out_type=[
            jax.ShapeDtypeStruct((B, NBINS // 64, 128), jnp.float32),
            jax.ShapeDtypeStruct((B, L), jnp.float32),
            jax.ShapeDtypeStruct((B, L), jnp.float32),
        ],
        mesh=mesh,
        compiler_params=pltpu.CompilerParams(needs_layout_passes=False),
        scratch_types=[
            pltpu.VMEM((3, GPT, 1, 128), jnp.float32),
            pltpu.VMEM((NROW, 128), jnp.int32),
            pltpu.VMEM((128,), jnp.float32),
            pltpu.VMEM((BINS_PT,), jnp.float32),
            pltpu.VMEM((2, 64, 128), jnp.float32),
            pltpu.VMEM((ZB,), jnp.float32),
            pltpu.VMEM((L,), jnp.float32),
            pltpu.VMEM((L,), jnp.int32),
            pltpu.VMEM((SLOT,), jnp.float32),
            pltpu.VMEM_SHARED((NBINS,), jnp.float32),
            pltpu.VMEM_SHARED((NSLOT * SLOT,), jnp.float32),
            pltpu.SemaphoreType.DMA,
            pltpu.SemaphoreType.DMA,
        ],
    )
    return f(points4)


def kernel(points):
    points4 = (jnp.transpose(points, (2, 1, 0))
               .reshape(3, NG, 128, B)
               .transpose(0, 1, 3, 2))
    grid, minsb, rangesb = _voxelize(points4)
    return (grid[:, :, :64].reshape(B, 1, D, H, W),
            minsb[:, :3].reshape(B, 1, 3),
            rangesb[:, :3].reshape(B, 1, 3))


# speedup vs baseline: 1.0737x; 1.0737x over previous
import jax
import jax.numpy as jnp
from jax import lax
from jax.experimental import pallas as pl
from jax.experimental.pallas import tpu as pltpu
from jax.experimental.pallas import tpu_sc as plsc

B = 8
N = 262144
D = H = W = 64
NBINS = D * H * W
NC = 2
NS = 16
L = 16
BPC = B // NC
NG = N // 128
GPT = NG // NS
PPT = N // NS
NGRP = PPT // L
BINS_PT = NBINS // NS
NROW = PPT // 128
ZB = 8192
SLOT = NS * 64
NSLOT = BPC * 2
SCALE = float(64.0 - 1e-06)
ONE_M_EPS = float(1.0 - 1e-06)
CLIP2 = float((1.0 - 1e-06) * (64.0 - 1e-06))
INF = float("inf")


def _perm(v, idx):
    return v.at[idx].get(mode="promise_in_bounds", unique_indices=False)


def _bf_min(v, lane):
    for sh in (8, 4, 2, 1):
        v = jnp.minimum(v, _perm(v, lane ^ sh))
    return v


def _bf_max(v, lane):
    for sh in (8, 4, 2, 1):
        v = jnp.maximum(v, _perm(v, lane ^ sh))
    return v


def _body(points_ref, grid_ref, mins_ref, ranges_ref,
          stage, idxbuf, ones, hslice, zbuf, prow, pidx, pgridf, hist, mm,
          sem, sem2):
    c = lax.axis_index("c")
    s = lax.axis_index("s")
    lane = lax.broadcasted_iota(jnp.int32, (L,), 0)

    zeros16 = jnp.zeros((L,), jnp.float32)
    ones16 = jnp.full((L,), 1.0, jnp.float32)

    def fill(k, _):
        ones[pl.ds(k * L, L)] = ones16
        return 0

    def zfill(k, _):
        zbuf[pl.ds(k * L, L)] = zeros16
        return 0

    lax.fori_loop(0, 128 // L, fill, 0)
    lax.fori_loop(0, ZB // L, zfill, 0)
    for hb in range(2):
        for q in range(BINS_PT // ZB):
            pltpu.sync_copy(
                zbuf, hist.at[pl.ds(hb * NBINS + s * BINS_PT + q * ZB, ZB)])
    mm_share = NSLOT * SLOT // NS
    pltpu.sync_copy(zbuf.at[pl.ds(0, mm_share)],
                    mm.at[pl.ds(s * mm_share, mm_share)])
    plsc.subcore_barrier()

    def minmax_publish(bl):
        def mm_body(r, carry):
            accs = list(carry)
            for d in range(8):
                col = d * L
                for cc in range(3):
                    v = stage[cc, r, 0, pl.ds(col, L)]
                    accs[cc] = jnp.minimum(accs[cc], v)
                    accs[3 + cc] = jnp.maximum(accs[3 + cc], v)
            return tuple(accs)

        init = (jnp.full((L,), INF), jnp.full((L,), INF), jnp.full((L,), INF),
                jnp.full((L,), -INF), jnp.full((L,), -INF), jnp.full((L,), -INF))
        with jax.named_scope("minmax"):
            acc6 = lax.fori_loop(0, GPT, mm_body, init)

        row = jnp.full((L,), INF)
        for cc in range(3):
            row = jnp.where(lane == cc, _bf_min(acc6[cc], lane), row)
            row = jnp.where(lane == 3 + cc, -_bf_max(acc6[3 + cc], lane), row)
        prow[...] = row
        pidx[...] = (bl * 2) * SLOT + s * 64 + lane
        pltpu.sync_copy(prow, mm.at[pidx], add=True)

    with jax.named_scope("stagein"):
        cps0 = [pltpu.async_copy(
            points_ref.at[cc, pl.ds(s * GPT, GPT), pl.ds(c * BPC, 1), :],
            stage.at[cc], sem2) for cc in range(3)]
        for cp in cps0:
            cp.wait()
    minmax_publish(0)
    plsc.subcore_barrier()

    def phase3(bl):
        b = c * BPC + bl
        hb = (bl & 1) * NBINS
        pltpu.sync_copy(hist.at[pl.ds(hb + s * BINS_PT, BINS_PT)], hslice)
        for q in range(BINS_PT // ZB):
            pltpu.sync_copy(zbuf, hist.at[pl.ds(hb + s * BINS_PT + q * ZB, ZB)])

        def max_body(r, m):
            for d in range(8):
                m = jnp.maximum(m, hslice[pl.ds(r * 128 + d * L, L)])
            return m

        with jax.named_scope("hmax"):
            m = lax.fori_loop(0, BINS_PT // 128, max_body, jnp.full((L,), -INF))
        prow[...] = jnp.where(lane == 0, -_bf_max(m, lane), INF)
        pidx[...] = (bl * 2 + 1) * SLOT + s * 64 + lane
        pltpu.sync_copy(prow, mm.at[pidx], add=True)
        plsc.subcore_barrier()

        pltpu.sync_copy(mm.at[pl.ds((bl * 2 + 1) * SLOT, SLOT)], pgridf)
        acc = pgridf[pl.ds(0, L)]
        for t in range(1, NS):
            acc = jnp.minimum(acc, pgridf[pl.ds(t * 64, L)])
        gmaxb = -_perm(acc, jnp.zeros((L,), jnp.int32))
        invn = 1.0 / jnp.maximum(gmaxb, 1.0)

        def norm_body(r, _):
            for d in range(8):
                o = r * 128 + d * L
                hslice[pl.ds(o, L)] = hslice[pl.ds(o, L)] * invn
            return 0

        with jax.named_scope("norm"):
            lax.fori_loop(0, BINS_PT // 128, norm_body, 0)
        pltpu.sync_copy(hslice, grid_ref.at[b, pl.ds(s * BINS_PT, BINS_PT)])

    for bl in range(BPC):
        b = c * BPC + bl

        pltpu.sync_copy(mm.at[pl.ds((bl * 2) * SLOT, SLOT)], pgridf)
        acc = pgridf[pl.ds(0, L)]
        for t in range(1, NS):
            acc = jnp.minimum(acc, pgridf[pl.ds(t * 64, L)])
        mnb = [_perm(acc, jnp.full((L,), cc, jnp.int32)) for cc in range(3)]
        ngb = [_perm(acc, jnp.full((L,), 3 + cc, jnp.int32)) for cc in range(3)]
        rngb = [jnp.maximum(-ngb[cc] - mnb[cc], 1e-06) for cc in range(3)]
        invb = [(1.0 / rngb[cc]) * SCALE for cc in range(3)]

        @pl.when(s == 0)
        def _():
            prow[...] = jnp.where(lane == 0, mnb[0],
                                  jnp.where(lane == 1, mnb[1],
                                            jnp.where(lane == 2, mnb[2], 0.0)))
            pltpu.sync_copy(prow, mins_ref.at[b])
            prow[...] = jnp.where(lane == 0, rngb[0],
                                  jnp.where(lane == 1, rngb[1],
                                            jnp.where(lane == 2, rngb[2], 0.0)))
            pltpu.sync_copy(prow, ranges_ref.at[b])

        base = (bl & 1) * NBINS

        def row_body(j, _):
            for d in range(8):
                col = d * L
                ii = []
                for cc in range(3):
                    xs = stage[cc, j, 0, pl.ds(col, L)]
                    v = (xs - mnb[cc]) * invb[cc]
                    v = jnp.minimum(jnp.maximum(v, 0.0), CLIP2)
                    ii.append(v.astype(jnp.int32))
                linear = base + ii[0] * (H * W) + ii[1] * W + ii[2]
                idxbuf[j, pl.ds(col, L)] = linear
            pltpu.async_copy(ones, hist.at[idxbuf.at[j]], sem, add=True)
            return 0

        with jax.named_scope("idxsc"):
            lax.fori_loop(0, NROW, row_body, 0)

        if bl + 1 < BPC:
            with jax.named_scope("stagein"):
                cps = [pltpu.async_copy(
                    points_ref.at[cc, pl.ds(s * GPT, GPT), pl.ds(b + 1, 1), :],
                    stage.at[cc], sem2) for cc in range(3)]
        if bl > 0:
            phase3(bl - 1)
        if bl + 1 < BPC:
            for cp in cps:
                cp.wait()
            minmax_publish(bl + 1)

        with jax.named_scope("drain"):
            for j in range(NROW):
                pltpu.make_async_copy(ones, hist.at[idxbuf.at[0]], sem).wait()
        plsc.subcore_barrier()

    phase3(BPC - 1)


@jax.jit
def _voxelize(points4):
    mesh = plsc.VectorSubcoreMesh(core_axis_name="c", subcore_axis_name="s",
                                  num_cores=NC, num_subcores=NS)
    f = pl.kernel(
        _body,
        out_type=[
            jax.ShapeDtypeStruct((B, NBINS), jnp.float32),
            jax.ShapeDtypeStruct((B, L), jnp.float32),
            jax.ShapeDtypeStruct((B, L), jnp.float32),
        ],
        mesh=mesh,
        compiler_params=pltpu.CompilerParams(needs_layout_passes=False),
        scratch_types=[
            pltpu.VMEM((3, GPT, 1, 128), jnp.float32),
            pltpu.VMEM((NROW, 128), jnp.int32),
            pltpu.VMEM((128,), jnp.float32),
            pltpu.VMEM((BINS_PT,), jnp.float32),
            pltpu.VMEM((ZB,), jnp.float32),
            pltpu.VMEM((L,), jnp.float32),
            pltpu.VMEM((L,), jnp.int32),
            pltpu.VMEM((SLOT,), jnp.float32),
            pltpu.VMEM_SHARED((2 * NBINS,), jnp.float32),
            pltpu.VMEM_SHARED((NSLOT * SLOT,), jnp.float32),
            pltpu.SemaphoreType.DMA,
            pltpu.SemaphoreType.DMA,
        ],
    )
    return f(points4)


def kernel(points):
    points4 = (jnp.transpose(points, (2, 1, 0))
               .reshape(3, NG, 128, B)
               .transpose(0, 1, 3, 2))
    grid, minsb, rangesb = _voxelize(points4)
    return (grid.reshape(B, 1, D, H, W),
            minsb[:, :3].reshape(B, 1, 3),
            rangesb[:, :3].reshape(B, 1, 3))
